# baseline (speedup 1.0000x reference)
import functools

import jax
import jax.numpy as jnp
from jax import lax
from jax.experimental import pallas as pl
from jax.experimental.pallas import tpu as pltpu
from jax.experimental.pallas import tpu_sc as plsc

_NC = 2
_NS = 16
_L = 16
_NW = _NC * _NS
_B = 4096
_F = 32
_BPW = _B // _NW
_SLOTS = 8
_CHUNKS = _BPW // _SLOTS

_NI = 100000
_PACK_COLS = 512
_PACK_GRID = -(-_NI // _PACK_COLS)
_PACK_ROWS = _PACK_GRID * 128

_ROWS_PER_BLK = 512


def _sc_gather_dot(user, item, uft, itt, ub_flat, ib_flat):
    mesh = plsc.VectorSubcoreMesh(
        core_axis_name="c", subcore_axis_name="s",
        num_cores=_NC, num_subcores=_NS)

    @functools.partial(
        pl.kernel,
        out_type=(
            jax.ShapeDtypeStruct((_B,), jnp.float32),
            jax.ShapeDtypeStruct((_B,), jnp.float32),
        ),
        mesh=mesh,
        compiler_params=pltpu.CompilerParams(needs_layout_passes=False),
        scratch_types=[
            pltpu.VMEM((_BPW + 2 * _SLOTS,), jnp.int32),
            pltpu.VMEM((_BPW + 2 * _SLOTS,), jnp.int32),
            pltpu.VMEM((_SLOTS, _F, 128), jnp.float32),
            pltpu.VMEM((_SLOTS, _F, 128), jnp.float32),
            pltpu.VMEM((_F * _BPW,), jnp.float32),
            pltpu.VMEM((_F * _BPW,), jnp.float32),
            pltpu.VMEM((_BPW,), jnp.float32),
            pltpu.VMEM((_BPW,), jnp.float32),
            pltpu.VMEM((_BPW,), jnp.float32),
            pltpu.VMEM((_BPW,), jnp.float32),
        ] + [pltpu.SemaphoreType.DMA] * (2 * _SLOTS + 2),
    )
    def sc_kernel(user_hbm, item_hbm, uft_hbm, itt_hbm, ub_hbm, ib_hbm,
                  dot_hbm, row_hbm,
                  uidx_v, iidx_v, ubuf, ibuf, ufc_v, itc_v,
                  ubg_v, ibg_v, dot_v, row_v, *sems):
        wid = lax.axis_index("s") * _NC + lax.axis_index("c")
        base = wid * _BPW
        pltpu.sync_copy(user_hbm.at[pl.ds(base, _BPW)],
                        uidx_v.at[pl.ds(0, _BPW)])
        pltpu.sync_copy(item_hbm.at[pl.ds(base, _BPW)],
                        iidx_v.at[pl.ds(0, _BPW)])

        cb0 = pltpu.async_copy(ub_hbm.at[uidx_v.at[pl.ds(0, _BPW)]],
                               ubg_v, sems[2 * _SLOTS])
        cb1 = pltpu.async_copy(ib_hbm.at[iidx_v.at[pl.ds(0, _BPW)]],
                               ibg_v, sems[2 * _SLOTS + 1])

        lanes = lax.iota(jnp.int32, _L)

        def fetch(tbl_hbm, buf, slot, sem, rid):
            off = pl.multiple_of((rid >> 7) << 7, 128)
            pltpu.async_copy(tbl_hbm.at[pl.ds(0, _F), pl.ds(off, 128)],
                             buf.at[slot], sem)

        vec_u0 = uidx_v[pl.ds(0, _L)]
        vec_i0 = iidx_v[pl.ds(0, _L)]
        for l in range(_SLOTS):
            fetch(uft_hbm, ubuf, l, sems[l], vec_u0[l])
            fetch(itt_hbm, ibuf, l, sems[_SLOTS + l], vec_i0[l])

        def extract(buf, slot, dst, q, b_vec):
            q_vec = jnp.full((_L,), q, jnp.int32)
            for h in range(2):
                v = plsc.load_gather(
                    buf.at[slot], [lanes + h * _L, q_vec])
                plsc.store_scatter(
                    dst, [(lanes + h * _L) * _BPW + b_vec], v)

        def chunk(c, carry):
            vec_u = uidx_v[pl.ds(c * _SLOTS, 2 * _SLOTS)]
            vec_i = iidx_v[pl.ds(c * _SLOTS, 2 * _SLOTS)]
            for l in range(_SLOTS):
                b = c * _SLOTS + l
                b_vec = jnp.full((_L,), b, jnp.int32)
                pltpu.make_async_copy(
                    uft_hbm.at[pl.ds(0, _F), pl.ds(0, 128)],
                    ubuf.at[l], sems[l]).wait()
                extract(ubuf, l, ufc_v, vec_u[l] & 127, b_vec)
                pltpu.make_async_copy(
                    itt_hbm.at[pl.ds(0, _F), pl.ds(0, 128)],
                    ibuf.at[l], sems[_SLOTS + l]).wait()
                extract(ibuf, l, itc_v, vec_i[l] & 127, b_vec)

                @pl.when(c < _CHUNKS - 1)
                def _():
                    fetch(uft_hbm, ubuf, l, sems[l], vec_u[_SLOTS + l])
                    fetch(itt_hbm, ibuf, l, sems[_SLOTS + l],
                          vec_i[_SLOTS + l])
            return carry

        lax.fori_loop(0, _CHUNKS, chunk, 0)
        cb0.wait()
        cb1.wait()

        for c8 in range(_BPW // _L):
            sl0 = c8 * _L
            acc = (ufc_v[pl.ds(sl0, _L)] * itc_v[pl.ds(sl0, _L)])
            for f in range(1, _F):
                acc = acc + (ufc_v[pl.ds(f * _BPW + sl0, _L)] *
                             itc_v[pl.ds(f * _BPW + sl0, _L)])
            dot_v[pl.ds(sl0, _L)] = acc
            row_v[pl.ds(sl0, _L)] = (ubg_v[pl.ds(sl0, _L)] +
                                     ibg_v[pl.ds(sl0, _L)])

        pltpu.sync_copy(dot_v, dot_hbm.at[pl.ds(base, _BPW)])
        pltpu.sync_copy(row_v, row_hbm.at[pl.ds(base, _BPW)])

    return sc_kernel(user, item, uft, itt, ub_flat, ib_flat)


def _dense_body(rt_ref, dot_ref, out_ref):
    out_ref[...] = jax.nn.sigmoid(rt_ref[...] + dot_ref[...])


def _tc_dense(rowterm, dot):
    rt2 = rowterm.reshape(_B, 1)
    dot2 = dot.reshape(1, _B)
    grid = (_B // _ROWS_PER_BLK,)
    return pl.pallas_call(
        _dense_body,
        out_shape=jax.ShapeDtypeStruct((_B, _B), jnp.float32),
        grid=grid,
        in_specs=[
            pl.BlockSpec((_ROWS_PER_BLK, 1), lambda i: (i, 0)),
            pl.BlockSpec((1, _B), lambda i: (0, 0)),
        ],
        out_specs=pl.BlockSpec((_ROWS_PER_BLK, _B), lambda i: (i, 0)),
    )(rt2, dot2)


def kernel(user, item, user_factors, item_factors, user_bias, item_bias):
    dot, rowterm = _sc_gather_dot(
        user, item, user_factors.T, item_factors.T,
        lax.squeeze(user_bias, (1,)), lax.squeeze(item_bias, (1,)))
    return _tc_dense(rowterm, dot)


# speedup vs baseline: 1.0008x; 1.0008x over previous
import functools

import jax
import jax.numpy as jnp
from jax import lax
from jax.experimental import pallas as pl
from jax.experimental.pallas import tpu as pltpu
from jax.experimental.pallas import tpu_sc as plsc

_NC = 2
_NS = 16
_L = 16
_NW = _NC * _NS
_B = 4096
_F = 32
_BPW = _B // _NW
_SLOTS = 8
_CHUNKS = _BPW // _SLOTS

_ROWS_PER_BLK = 512


def _sc_gather_dot(user, item, uft, itt, ub_flat, ib_flat):
    mesh = plsc.VectorSubcoreMesh(
        core_axis_name="c", subcore_axis_name="s",
        num_cores=_NC, num_subcores=_NS)

    @functools.partial(
        pl.kernel,
        out_type=(
            jax.ShapeDtypeStruct((_B,), jnp.float32),
            jax.ShapeDtypeStruct((_B,), jnp.float32),
        ),
        mesh=mesh,
        compiler_params=pltpu.CompilerParams(needs_layout_passes=False),
        scratch_types=[
            pltpu.VMEM((_BPW + 2 * _SLOTS,), jnp.int32),
            pltpu.VMEM((_BPW + 2 * _SLOTS,), jnp.int32),
            pltpu.VMEM((_SLOTS, _F, 128), jnp.float32),
            pltpu.VMEM((_SLOTS, _F, 128), jnp.float32),
            pltpu.VMEM((_F * _BPW,), jnp.float32),
            pltpu.VMEM((_F * _BPW,), jnp.float32),
            pltpu.VMEM((_BPW,), jnp.float32),
            pltpu.VMEM((_BPW,), jnp.float32),
            pltpu.VMEM((_BPW,), jnp.float32),
            pltpu.VMEM((_BPW,), jnp.float32),
        ] + [pltpu.SemaphoreType.DMA] * (2 * _SLOTS + 2),
    )
    def sc_kernel(user_hbm, item_hbm, uft_hbm, itt_hbm, ub_hbm, ib_hbm,
                  dot_hbm, row_hbm,
                  uidx_v, iidx_v, ubuf, ibuf, ufc_v, itc_v,
                  ubg_v, ibg_v, dot_v, row_v, *sems):
        wid = lax.axis_index("s") * _NC + lax.axis_index("c")
        base = wid * _BPW
        pltpu.sync_copy(user_hbm.at[pl.ds(base, _BPW)],
                        uidx_v.at[pl.ds(0, _BPW)])
        pltpu.sync_copy(item_hbm.at[pl.ds(base, _BPW)],
                        iidx_v.at[pl.ds(0, _BPW)])

        cb0 = pltpu.async_copy(ub_hbm.at[uidx_v.at[pl.ds(0, _BPW)]],
                               ubg_v, sems[2 * _SLOTS])
        cb1 = pltpu.async_copy(ib_hbm.at[iidx_v.at[pl.ds(0, _BPW)]],
                               ibg_v, sems[2 * _SLOTS + 1])

        lanes = lax.iota(jnp.int32, _L)

        def fetch(tbl_hbm, buf, slot, sem, rid):
            off = pl.multiple_of((rid >> 7) << 7, 128)
            pltpu.async_copy(tbl_hbm.at[pl.ds(0, _F), pl.ds(off, 128)],
                             buf.at[slot], sem)

        vec_u0 = uidx_v[pl.ds(0, _L)]
        vec_i0 = iidx_v[pl.ds(0, _L)]
        for l in range(_SLOTS):
            fetch(uft_hbm, ubuf, l, sems[l], vec_u0[l])
            fetch(itt_hbm, ibuf, l, sems[_SLOTS + l], vec_i0[l])

        def extract(buf, slot, dst, q, b_vec):
            q_vec = jnp.full((_L,), q, jnp.int32)
            for h in range(2):
                v = plsc.load_gather(
                    buf.at[slot], [lanes + h * _L, q_vec])
                plsc.store_scatter(
                    dst, [(lanes + h * _L) * _BPW + b_vec], v)

        def chunk(c, carry):
            vec_u = uidx_v[pl.ds(c * _SLOTS, 2 * _SLOTS)]
            vec_i = iidx_v[pl.ds(c * _SLOTS, 2 * _SLOTS)]
            for l in range(_SLOTS):
                b = c * _SLOTS + l
                b_vec = jnp.full((_L,), b, jnp.int32)
                pltpu.make_async_copy(
                    uft_hbm.at[pl.ds(0, _F), pl.ds(0, 128)],
                    ubuf.at[l], sems[l]).wait()
                extract(ubuf, l, ufc_v, vec_u[l] & 127, b_vec)
                pltpu.make_async_copy(
                    itt_hbm.at[pl.ds(0, _F), pl.ds(0, 128)],
                    ibuf.at[l], sems[_SLOTS + l]).wait()
                extract(ibuf, l, itc_v, vec_i[l] & 127, b_vec)

                @pl.when(c < _CHUNKS - 1)
                def _():
                    fetch(uft_hbm, ubuf, l, sems[l], vec_u[_SLOTS + l])
                    fetch(itt_hbm, ibuf, l, sems[_SLOTS + l],
                          vec_i[_SLOTS + l])
            return carry

        lax.fori_loop(0, _CHUNKS, chunk, 0)
        cb0.wait()
        cb1.wait()

        for c8 in range(_BPW // _L):
            sl0 = c8 * _L
            acc = (ufc_v[pl.ds(sl0, _L)] * itc_v[pl.ds(sl0, _L)])
            for f in range(1, _F):
                acc = acc + (ufc_v[pl.ds(f * _BPW + sl0, _L)] *
                             itc_v[pl.ds(f * _BPW + sl0, _L)])
            dot_v[pl.ds(sl0, _L)] = acc
            row_v[pl.ds(sl0, _L)] = (ubg_v[pl.ds(sl0, _L)] +
                                     ibg_v[pl.ds(sl0, _L)])

        pltpu.sync_copy(dot_v, dot_hbm.at[pl.ds(base, _BPW)])
        pltpu.sync_copy(row_v, row_hbm.at[pl.ds(base, _BPW)])

    return sc_kernel(user, item, uft, itt, ub_flat, ib_flat)


def _dense_body(rt_ref, dot_ref, out_ref):
    out_ref[...] = jax.nn.sigmoid(rt_ref[...] + dot_ref[...])


def _tc_dense(rowterm, dot):
    rt2 = rowterm.reshape(_B, 1)
    dot2 = dot.reshape(1, _B)
    grid = (_B // _ROWS_PER_BLK,)
    return pl.pallas_call(
        _dense_body,
        out_shape=jax.ShapeDtypeStruct((_B, _B), jnp.float32),
        grid=grid,
        in_specs=[
            pl.BlockSpec((_ROWS_PER_BLK, 1), lambda i: (i, 0)),
            pl.BlockSpec((1, _B), lambda i: (0, 0)),
        ],
        out_specs=pl.BlockSpec((_ROWS_PER_BLK, _B), lambda i: (i, 0)),
    )(rt2, dot2)


def kernel(user, item, user_factors, item_factors, user_bias, item_bias):
    dot, rowterm = _sc_gather_dot(
        user, item, user_factors.T, item_factors.T,
        lax.squeeze(user_bias, (1,)), lax.squeeze(item_bias, (1,)))
    return _tc_dense(rowterm, dot)
